# speedup vs baseline: 1.8208x; 1.8208x over previous
import functools

import jax
import jax.numpy as jnp
from jax import lax
from jax.experimental import pallas as pl
from jax.experimental.pallas import tpu as pltpu
from jax.experimental.pallas import tpu_sc as plsc

N_TOK = 32768
D = 768
LANES = 16
NC = 2
NS = 16
NW = NC * NS
TPW = N_TOK // NW
C = 32
NCHUNK = TPW // C


def _affine_gate(x, idx, gamma, beta):
    mesh = plsc.VectorSubcoreMesh(core_axis_name="c", subcore_axis_name="s")

    @functools.partial(
        pl.kernel,
        mesh=mesh,
        out_type=jax.ShapeDtypeStruct((N_TOK, D), jnp.float32),
        scratch_types=[
            pltpu.VMEM((C,), jnp.int32),
            pltpu.VMEM((C, D), jnp.float32),
            pltpu.VMEM((C, D), jnp.float32),
            pltpu.VMEM((C, D), jnp.float32),
            pltpu.SemaphoreType.DMA,
        ],
    )
    def k(x_hbm, idx_hbm, gamma_hbm, beta_hbm, out_hbm, idx_v, x_v, g_v, b_v, sem):
        wid = lax.axis_index("s") * NC + lax.axis_index("c")
        wbase = wid * TPW

        def chunk_body(ci, carry):
            base = wbase + ci * C
            pltpu.sync_copy(idx_hbm.at[pl.ds(base, C)], idx_v)
            pltpu.sync_copy(x_hbm.at[pl.ds(base, C)], x_v)
            pltpu.async_copy(gamma_hbm.at[idx_v], g_v, sem).wait()
            pltpu.async_copy(beta_hbm.at[idx_v], b_v, sem).wait()

            def tok_body(t, tc):
                for d in range(D // LANES):
                    sl = pl.ds(d * LANES, LANES)
                    x_v[t, sl] = x_v[t, sl] * g_v[t, sl] + b_v[t, sl]
                return tc

            lax.fori_loop(0, C, tok_body, 0)
            pltpu.sync_copy(x_v, out_hbm.at[pl.ds(base, C)])
            return carry

        lax.fori_loop(0, NCHUNK, chunk_body, 0)

    return k(x, idx, gamma, beta)


def kernel(x, group, gamma, beta):
    idx = group.reshape(-1)
    return _affine_gate(x, idx, gamma, beta)


# speedup vs baseline: 2.9611x; 1.6263x over previous
import functools

import jax
import jax.numpy as jnp
from jax import lax
from jax.experimental import pallas as pl
from jax.experimental.pallas import tpu as pltpu
from jax.experimental.pallas import tpu_sc as plsc

N_TOK = 32768
D = 768
LANES = 16
NC = 2
NS = 16
NW = NC * NS
TPW = N_TOK // NW
C = 32
NCHUNK = TPW // C
NBUF = 4


def _affine_gate(x, idx, gamma, beta):
    mesh = plsc.VectorSubcoreMesh(core_axis_name="c", subcore_axis_name="s")

    scratch = [
        pltpu.VMEM((8, D), jnp.float32),
        pltpu.VMEM((8, D), jnp.float32),
        pltpu.VMEM((NBUF, C, D), jnp.float32),
        pltpu.VMEM((NBUF, C), jnp.int32),
    ]
    scratch += [pltpu.SemaphoreType.DMA] * (3 * NBUF)

    @functools.partial(
        pl.kernel,
        mesh=mesh,
        out_type=jax.ShapeDtypeStruct((N_TOK, D), jnp.float32),
        scratch_types=scratch,
    )
    def k(x_hbm, idx_hbm, gamma_hbm, beta_hbm, out_hbm, gtab, btab, xb, ib,
          *sems):
        s_x = sems[0:NBUF]
        s_i = sems[NBUF:2 * NBUF]
        s_o = sems[2 * NBUF:3 * NBUF]
        wid = lax.axis_index("s") * NC + lax.axis_index("c")
        wbase = wid * TPW

        def start_in(b, ci):
            base = wbase + ci * C
            pltpu.make_async_copy(
                x_hbm.at[pl.ds(base, C)], xb.at[b], s_x[b]).start()
            pltpu.make_async_copy(
                idx_hbm.at[pl.ds(base, C)], ib.at[b], s_i[b]).start()

        def wait_in(b):
            pltpu.make_async_copy(
                x_hbm.at[pl.ds(0, C)], xb.at[b], s_x[b]).wait()
            pltpu.make_async_copy(
                idx_hbm.at[pl.ds(0, C)], ib.at[b], s_i[b]).wait()

        def start_out(b, ci):
            base = wbase + ci * C
            pltpu.make_async_copy(
                xb.at[b], out_hbm.at[pl.ds(base, C)], s_o[b]).start()

        def wait_out(b):
            pltpu.make_async_copy(
                xb.at[b], out_hbm.at[pl.ds(0, C)], s_o[b]).wait()

        pltpu.sync_copy(gamma_hbm, gtab)
        pltpu.sync_copy(beta_hbm, btab)

        for b in range(NBUF):
            start_in(b, b)

        def compute(b):
            def tg_body(tg, tc):
                t0 = tg * LANES
                ev = ib[b, pl.ds(t0, LANES)]
                es = [ev[j] for j in range(LANES)]

                def d_body(d, dc):
                    sl = pl.ds(d * LANES, LANES)
                    for j in range(LANES):
                        t = t0 + j
                        xb[b, t, sl] = (
                            xb[b, t, sl] * gtab[es[j], sl] + btab[es[j], sl])
                    return dc

                lax.fori_loop(0, D // LANES, d_body, 0)
                return tc

            lax.fori_loop(0, C // LANES, tg_body, 0)

        def group_body(g0, carry):
            for b in range(NBUF):
                ci = g0 * NBUF + b
                wait_in(b)
                compute(b)
                start_out(b, ci)
                bp = (b - 1) % NBUF
                cip = ci - 1 + NBUF

                @pl.when(jnp.logical_and(ci >= 1, cip < NCHUNK))
                def _():
                    wait_out(bp)
                    start_in(bp, cip)

            return carry

        lax.fori_loop(0, NCHUNK // NBUF, group_body, 0)

        for b in range(NBUF):
            wait_out(b)

    return k(x, idx, gamma, beta)


def kernel(x, group, gamma, beta):
    idx = group.reshape(-1)
    return _affine_gate(x, idx, gamma, beta)
